# baseline (speedup 1.0000x reference)
import functools

import jax
import jax.numpy as jnp
from jax import lax
from jax.experimental import pallas as pl
from jax.experimental.pallas import tpu as pltpu
from jax.experimental.pallas import tpu_sc as plsc

N = 100000
D = 128
E = 200000
T = 7

R = 2000
RP = 2048
P = 25
G = 64
PAD = R
DW = 16

EP = 200704
SL = 12544
CH = 1792
MF = 12672


def _tc_transform(x, Ws):
    BN = 400

    def body(x_ref, w_ref, o_ref):
        o_ref[0] = jnp.dot(x_ref[...], w_ref[0],
                           preferred_element_type=jnp.float32)

    return pl.pallas_call(
        body,
        grid=(N // BN, T),
        in_specs=[
            pl.BlockSpec((BN, D), lambda i, t: (i, 0)),
            pl.BlockSpec((1, D, D), lambda i, t: (t, 0, 0)),
        ],
        out_specs=pl.BlockSpec((1, BN, D), lambda i, t: (t, i, 0)),
        out_shape=jax.ShapeDtypeStruct((T, N, D), jnp.float32),
    )(x, Ws)


def _sc_body(y_hbm, src_hbm, dst_hbm, bbar_hbm, out_hbm,
             src_buf, dst_buf, msrc_f, moff_f, moff_blk, rows_buf, ones_buf,
             inv_buf, pfx_buf, scale_buf, zbuf, zdeg, dstage, bbar_buf,
             wout_buf, acc_sp, deg_sp, sem):
    cid = lax.axis_index("c")
    sid = lax.axis_index("s")

    z16f = jnp.zeros((16,), jnp.float32)
    z16i = jnp.zeros((16,), jnp.int32)
    one16 = jnp.ones((16,), jnp.float32)
    pad16 = jnp.full((16,), PAD, jnp.int32)
    lanes = lax.iota(jnp.int32, 16)

    def _init_zb(r, _):
        for k in range(8):
            zbuf[r, pl.ds(16 * k, 16)] = z16f
        return 0
    lax.fori_loop(0, 16, _init_zb, 0)

    def _init_zd(r, _):
        zdeg[r, pl.ds(0, 16)] = z16f
        return 0
    lax.fori_loop(0, 128, _init_zd, 0)

    def _init_ms(i, _):
        msrc_f[pl.ds(16 * i, 16)] = z16i
        return 0
    lax.fori_loop(0, MF // 16, _init_ms, 0)

    def _init_on(r, _):
        ones_buf[r, pl.ds(0, 16)] = one16
        return 0
    lax.fori_loop(0, G, _init_on, 0)

    pltpu.sync_copy(bbar_hbm, bbar_buf)

    pltpu.sync_copy(zdeg, deg_sp.at[pl.ds(sid * 128, 128)])
    plsc.subcore_barrier()

    def scan_chunk(t, ebase, cnt0, base_node):
        pltpu.sync_copy(src_hbm.at[t].at[pl.ds(ebase, CH)], src_buf)
        pltpu.sync_copy(dst_hbm.at[t].at[pl.ds(ebase, CH)], dst_buf)

        def grp(g, cnt):
            sv = src_buf[pl.ds(16 * g, 16)]
            dv = dst_buf[pl.ds(16 * g, 16)]
            off = dv - base_node
            m = (off >= 0) & (off < R)
            v = m.astype(jnp.int32)
            mi = v
            for k in (1, 2, 4, 8):
                pfx_buf[...] = v
                sh = plsc.load_gather(pfx_buf, [jnp.maximum(lanes - k, 0)])
                v = v + jnp.where(lanes >= k, sh, 0)
            pos = jnp.where(m, cnt + v - mi, (MF - 16) + lanes)
            plsc.store_scatter(msrc_f, [pos], sv)
            plsc.store_scatter(moff_f, [pos], off)
            return cnt + v[15]

        return lax.fori_loop(0, CH // 16, grp, cnt0)

    def type_body(t, p_base):
        base_node = p_base

        def mm(i, _):
            moff_f[pl.ds(16 * i, 16)] = pad16
            return 0
        lax.fori_loop(0, MF // 16, mm, 0)

        ebase = sid * SL

        def sc(c, cnt):
            return scan_chunk(t, ebase + c * CH, cnt, base_node)
        cnt = lax.fori_loop(0, SL // CH, sc, 0)

        nblk = (cnt + (G - 1)) >> 6

        def dblk(b, _):
            def cp(k, _):
                moff_blk[pl.ds(16 * k, 16)] = moff_f[pl.ds(b * G + 16 * k, 16)]
                return 0
            lax.fori_loop(0, G // 16, cp, 0)
            pltpu.sync_copy(ones_buf, deg_sp.at[moff_blk], add=True)
            return 0
        lax.fori_loop(0, nblk, dblk, 0)

        plsc.subcore_barrier()

        def ic(c, _):
            pltpu.sync_copy(deg_sp.at[pl.ds(c * 128, 128)], dstage)

            def iv(i, _):
                dv = plsc.load_gather(dstage, [16 * i + lanes, z16i])
                inv_buf[pl.ds(c * 128 + 16 * i, 16)] = (
                    (1.0 / T) / jnp.maximum(dv, 1.0))
                return 0
            lax.fori_loop(0, 8, iv, 0)
            return 0
        lax.fori_loop(0, RP // 128, ic, 0)

        plsc.subcore_barrier()

        pltpu.sync_copy(zdeg, deg_sp.at[pl.ds(sid * 128, 128)])

        def ablk(b, _):
            def cp(k, _):
                moff_blk[pl.ds(16 * k, 16)] = moff_f[pl.ds(b * G + 16 * k, 16)]
                return 0
            lax.fori_loop(0, G // 16, cp, 0)
            pltpu.async_copy(y_hbm.at[t].at[msrc_f.at[pl.ds(b * G, G)]],
                             rows_buf, sem).wait()

            def sj(j, _):
                ivv = plsc.load_gather(inv_buf, [moff_blk[pl.ds(16 * j, 16)]])
                scale_buf[...] = ivv

                def se(e, _):
                    sp = plsc.load_gather(scale_buf,
                                          [jnp.full((16,), e, jnp.int32)])
                    r = 16 * j + e
                    for k in range(8):
                        rows_buf[r, pl.ds(16 * k, 16)] = (
                            rows_buf[r, pl.ds(16 * k, 16)] * sp)
                    return 0
                lax.fori_loop(0, 16, se, 0)
                return 0
            lax.fori_loop(0, G // 16, sj, 0)

            pltpu.sync_copy(rows_buf, acc_sp.at[moff_blk], add=True)
            return 0
        lax.fori_loop(0, nblk, ablk, 0)

        plsc.subcore_barrier()
        return p_base

    def pass_body(p, _):
        base_node = cid * (P * R) + p * R

        def zacc(kk, _):
            pltpu.sync_copy(zbuf, acc_sp.at[pl.ds(sid * 128 + kk * 16, 16)])
            return 0
        lax.fori_loop(0, 8, zacc, 0)

        plsc.subcore_barrier()

        lax.fori_loop(0, T, type_body, base_node)

        plsc.subcore_barrier()

        def wo(w, _):
            idx = sid + 16 * w

            @pl.when(idx < 125)
            def _():
                row0 = idx * 16
                pltpu.sync_copy(acc_sp.at[pl.ds(row0, 16)], wout_buf)

                def wr(r, _):
                    for k in range(8):
                        wout_buf[r, pl.ds(16 * k, 16)] = (
                            wout_buf[r, pl.ds(16 * k, 16)]
                            + bbar_buf[pl.ds(16 * k, 16)])
                    return 0
                lax.fori_loop(0, 16, wr, 0)
                pltpu.sync_copy(wout_buf,
                                out_hbm.at[pl.ds(base_node + row0, 16)])
            return 0
        lax.fori_loop(0, 8, wo, 0)

        plsc.subcore_barrier()
        return 0

    lax.fori_loop(0, P, pass_body, 0)


def _sc_aggregate(y, src_all, dst_all, bbar):
    mesh = plsc.VectorSubcoreMesh(core_axis_name="c", subcore_axis_name="s")
    kfn = pl.kernel(
        _sc_body,
        out_type=jax.ShapeDtypeStruct((N, D), jnp.float32),
        mesh=mesh,
        compiler_params=pltpu.CompilerParams(needs_layout_passes=False),
        scratch_types=[
            pltpu.VMEM((CH,), jnp.int32),
            pltpu.VMEM((CH,), jnp.int32),
            pltpu.VMEM((MF,), jnp.int32),
            pltpu.VMEM((MF,), jnp.int32),
            pltpu.VMEM((G,), jnp.int32),
            pltpu.VMEM((G, D), jnp.float32),
            pltpu.VMEM((G, DW), jnp.float32),
            pltpu.VMEM((RP,), jnp.float32),
            pltpu.VMEM((16,), jnp.int32),
            pltpu.VMEM((16,), jnp.float32),
            pltpu.VMEM((16, D), jnp.float32),
            pltpu.VMEM((128, DW), jnp.float32),
            pltpu.VMEM((128, DW), jnp.float32),
            pltpu.VMEM((D,), jnp.float32),
            pltpu.VMEM((16, D), jnp.float32),
            pltpu.VMEM_SHARED((RP, D), jnp.float32),
            pltpu.VMEM_SHARED((RP, DW), jnp.float32),
            pltpu.SemaphoreType.DMA,
        ],
    )
    return kfn(y, src_all, dst_all, bbar)


def kernel(x,
           edge_index_candidate2candidate, W_candidate2candidate, b_candidate2candidate,
           edge_index_candidate2document, W_candidate2document, b_candidate2document,
           edge_index_candidate2entity, W_candidate2entity, b_candidate2entity,
           edge_index_codocument, W_codocument, b_codocument,
           edge_index_comention, W_comention, b_comention,
           edge_index_document2entity, W_document2entity, b_document2entity,
           edge_index_entity, W_entity, b_entity):
    edges = [edge_index_candidate2candidate, edge_index_candidate2document,
             edge_index_candidate2entity, edge_index_codocument,
             edge_index_comention, edge_index_document2entity,
             edge_index_entity]
    Ws = jnp.stack([W_candidate2candidate, W_candidate2document,
                    W_candidate2entity, W_codocument, W_comention,
                    W_document2entity, W_entity])
    bs = [b_candidate2candidate, b_candidate2document, b_candidate2entity,
          b_codocument, b_comention, b_document2entity, b_entity]

    bbar = sum(bs[1:], bs[0]) * (1.0 / T)
    src_all = jnp.stack([e[0] for e in edges])
    dst_all = jnp.stack([e[1] for e in edges])
    src_all = jnp.concatenate(
        [src_all, jnp.zeros((T, EP - E), jnp.int32)], axis=1)
    dst_all = jnp.concatenate(
        [dst_all, jnp.full((T, EP - E), -1, jnp.int32)], axis=1)

    y = _tc_transform(x, Ws)
    return _sc_aggregate(y, src_all, dst_all, bbar)


# speedup vs baseline: 1.1081x; 1.1081x over previous
import functools

import jax
import jax.numpy as jnp
from jax import lax
from jax.experimental import pallas as pl
from jax.experimental.pallas import tpu as pltpu
from jax.experimental.pallas import tpu_sc as plsc

N = 100000
D = 128
E = 200000
T = 7

R = 2000
RP = 2048
P = 25
G = 64
PAD = R
DW = 16

EP = 200704
SL = 12544
CH = 1792
MF = 12672


def _tc_transform(x, Ws):
    BN = 400

    def body(x_ref, w_ref, o_ref):
        o_ref[0] = jnp.dot(x_ref[...], w_ref[0],
                           preferred_element_type=jnp.float32)

    return pl.pallas_call(
        body,
        grid=(N // BN, T),
        in_specs=[
            pl.BlockSpec((BN, D), lambda i, t: (i, 0)),
            pl.BlockSpec((1, D, D), lambda i, t: (t, 0, 0)),
        ],
        out_specs=pl.BlockSpec((1, BN, D), lambda i, t: (t, i, 0)),
        out_shape=jax.ShapeDtypeStruct((T, N, D), jnp.float32),
    )(x, Ws)


def _sc_body(y_hbm, src_hbm, dst_hbm, bbar_hbm, out_hbm,
             src_buf, dst_buf, msrc_f, moff_f, moff_blk, rows_buf, ones_buf,
             inv_buf, pfx_buf, scale_buf, zbuf, zdeg, dstage, bbar_buf,
             wout_buf, acc_sp, deg_sp, sem, sem2):
    cid = lax.axis_index("c")
    sid = lax.axis_index("s")

    z16f = jnp.zeros((16,), jnp.float32)
    z16i = jnp.zeros((16,), jnp.int32)
    one16 = jnp.ones((16,), jnp.float32)
    pad16 = jnp.full((16,), PAD, jnp.int32)
    lanes = lax.iota(jnp.int32, 16)

    def _init_zb(r, _):
        for k in range(8):
            zbuf[r, pl.ds(16 * k, 16)] = z16f
        return 0
    lax.fori_loop(0, 16, _init_zb, 0)

    def _init_zd(r, _):
        zdeg[r, pl.ds(0, 16)] = z16f
        return 0
    lax.fori_loop(0, 128, _init_zd, 0)

    def _init_ms(i, _):
        msrc_f[pl.ds(16 * i, 16)] = z16i
        return 0
    lax.fori_loop(0, MF // 16, _init_ms, 0)

    def _init_on(r, _):
        ones_buf[r, pl.ds(0, 16)] = one16
        return 0
    lax.fori_loop(0, G, _init_on, 0)

    pltpu.sync_copy(bbar_hbm, bbar_buf)

    pltpu.sync_copy(zdeg, deg_sp.at[pl.ds(sid * 128, 128)])
    plsc.subcore_barrier()

    def scan_chunk(t, ebase, cnt0, base_node):
        c1 = pltpu.async_copy(src_hbm.at[t].at[pl.ds(ebase, CH)], src_buf,
                              sem)
        c2 = pltpu.async_copy(dst_hbm.at[t].at[pl.ds(ebase, CH)], dst_buf,
                              sem2)
        c1.wait()
        c2.wait()

        def grp(g, cnt):
            sv = src_buf[pl.ds(16 * g, 16)]
            dv = dst_buf[pl.ds(16 * g, 16)]
            off = dv - base_node
            m = (off >= 0) & (off < R)
            v = m.astype(jnp.int32)
            mi = v
            for k in (1, 2, 4, 8):
                pfx_buf[...] = v
                sh = plsc.load_gather(pfx_buf, [jnp.maximum(lanes - k, 0)])
                v = v + jnp.where(lanes >= k, sh, 0)
            pos = jnp.where(m, cnt + v - mi, (MF - 16) + lanes)
            plsc.store_scatter(msrc_f, [pos], sv)
            plsc.store_scatter(moff_f, [pos], off)
            return cnt + v[15]

        return lax.fori_loop(0, CH // 16, grp, cnt0)

    def type_body(t, p_base):
        base_node = p_base

        ebase = sid * SL

        def sc(c, cnt):
            return scan_chunk(t, ebase + c * CH, cnt, base_node)
        cnt = lax.fori_loop(0, SL // CH, sc, 0)

        for k in range(G // 16):
            plsc.store_scatter(moff_f, [cnt + 16 * k + lanes], pad16)

        nblk = (cnt + (G - 1)) >> 6

        def dblk(b, _):
            def cp(k, _):
                moff_blk[pl.ds(16 * k, 16)] = moff_f[pl.ds(b * G + 16 * k, 16)]
                return 0
            lax.fori_loop(0, G // 16, cp, 0)
            pltpu.sync_copy(ones_buf, deg_sp.at[moff_blk], add=True)
            return 0
        lax.fori_loop(0, nblk, dblk, 0)

        plsc.subcore_barrier()

        def ic(c, _):
            pltpu.sync_copy(deg_sp.at[pl.ds(c * 128, 128)], dstage)

            def iv(i, _):
                dv = plsc.load_gather(dstage, [16 * i + lanes, z16i])
                inv_buf[pl.ds(c * 128 + 16 * i, 16)] = (
                    (1.0 / T) / jnp.maximum(dv, 1.0))
                return 0
            lax.fori_loop(0, 8, iv, 0)
            return 0
        lax.fori_loop(0, RP // 128, ic, 0)

        plsc.subcore_barrier()

        pltpu.sync_copy(zdeg, deg_sp.at[pl.ds(sid * 128, 128)])

        def ablk(b, _):
            def cp(k, _):
                moff_blk[pl.ds(16 * k, 16)] = moff_f[pl.ds(b * G + 16 * k, 16)]
                return 0
            lax.fori_loop(0, G // 16, cp, 0)
            pltpu.async_copy(y_hbm.at[t].at[msrc_f.at[pl.ds(b * G, G)]],
                             rows_buf, sem).wait()

            def sj(j, _):
                ivv = plsc.load_gather(inv_buf, [moff_blk[pl.ds(16 * j, 16)]])
                scale_buf[...] = ivv

                def se(e, _):
                    sp = plsc.load_gather(scale_buf,
                                          [jnp.full((16,), e, jnp.int32)])
                    r = 16 * j + e
                    for k in range(8):
                        rows_buf[r, pl.ds(16 * k, 16)] = (
                            rows_buf[r, pl.ds(16 * k, 16)] * sp)
                    return 0
                lax.fori_loop(0, 16, se, 0)
                return 0
            lax.fori_loop(0, G // 16, sj, 0)

            pltpu.sync_copy(rows_buf, acc_sp.at[moff_blk], add=True)
            return 0
        lax.fori_loop(0, nblk, ablk, 0)

        plsc.subcore_barrier()
        return p_base

    def pass_body(p, _):
        base_node = cid * (P * R) + p * R

        def zacc(kk, _):
            pltpu.sync_copy(zbuf, acc_sp.at[pl.ds(sid * 128 + kk * 16, 16)])
            return 0
        lax.fori_loop(0, 8, zacc, 0)

        plsc.subcore_barrier()

        lax.fori_loop(0, T, type_body, base_node)

        plsc.subcore_barrier()

        def wo(w, _):
            idx = sid + 16 * w

            @pl.when(idx < 125)
            def _():
                row0 = idx * 16
                pltpu.sync_copy(acc_sp.at[pl.ds(row0, 16)], wout_buf)

                def wr(r, _):
                    for k in range(8):
                        wout_buf[r, pl.ds(16 * k, 16)] = (
                            wout_buf[r, pl.ds(16 * k, 16)]
                            + bbar_buf[pl.ds(16 * k, 16)])
                    return 0
                lax.fori_loop(0, 16, wr, 0)
                pltpu.sync_copy(wout_buf,
                                out_hbm.at[pl.ds(base_node + row0, 16)])
            return 0
        lax.fori_loop(0, 8, wo, 0)

        plsc.subcore_barrier()
        return 0

    lax.fori_loop(0, P, pass_body, 0)


def _sc_aggregate(y, src_all, dst_all, bbar):
    mesh = plsc.VectorSubcoreMesh(core_axis_name="c", subcore_axis_name="s")
    kfn = pl.kernel(
        _sc_body,
        out_type=jax.ShapeDtypeStruct((N, D), jnp.float32),
        mesh=mesh,
        compiler_params=pltpu.CompilerParams(needs_layout_passes=False),
        scratch_types=[
            pltpu.VMEM((CH,), jnp.int32),
            pltpu.VMEM((CH,), jnp.int32),
            pltpu.VMEM((MF,), jnp.int32),
            pltpu.VMEM((MF,), jnp.int32),
            pltpu.VMEM((G,), jnp.int32),
            pltpu.VMEM((G, D), jnp.float32),
            pltpu.VMEM((G, DW), jnp.float32),
            pltpu.VMEM((RP,), jnp.float32),
            pltpu.VMEM((16,), jnp.int32),
            pltpu.VMEM((16,), jnp.float32),
            pltpu.VMEM((16, D), jnp.float32),
            pltpu.VMEM((128, DW), jnp.float32),
            pltpu.VMEM((128, DW), jnp.float32),
            pltpu.VMEM((D,), jnp.float32),
            pltpu.VMEM((16, D), jnp.float32),
            pltpu.VMEM_SHARED((RP, D), jnp.float32),
            pltpu.VMEM_SHARED((RP, DW), jnp.float32),
            pltpu.SemaphoreType.DMA,
            pltpu.SemaphoreType.DMA,
        ],
    )
    return kfn(y, src_all, dst_all, bbar)


def kernel(x,
           edge_index_candidate2candidate, W_candidate2candidate, b_candidate2candidate,
           edge_index_candidate2document, W_candidate2document, b_candidate2document,
           edge_index_candidate2entity, W_candidate2entity, b_candidate2entity,
           edge_index_codocument, W_codocument, b_codocument,
           edge_index_comention, W_comention, b_comention,
           edge_index_document2entity, W_document2entity, b_document2entity,
           edge_index_entity, W_entity, b_entity):
    edges = [edge_index_candidate2candidate, edge_index_candidate2document,
             edge_index_candidate2entity, edge_index_codocument,
             edge_index_comention, edge_index_document2entity,
             edge_index_entity]
    Ws = jnp.stack([W_candidate2candidate, W_candidate2document,
                    W_candidate2entity, W_codocument, W_comention,
                    W_document2entity, W_entity])
    bs = [b_candidate2candidate, b_candidate2document, b_candidate2entity,
          b_codocument, b_comention, b_document2entity, b_entity]

    bbar = sum(bs[1:], bs[0]) * (1.0 / T)
    src_all = jnp.stack([e[0] for e in edges])
    dst_all = jnp.stack([e[1] for e in edges])
    src_all = jnp.concatenate(
        [src_all, jnp.zeros((T, EP - E), jnp.int32)], axis=1)
    dst_all = jnp.concatenate(
        [dst_all, jnp.full((T, EP - E), -1, jnp.int32)], axis=1)

    y = _tc_transform(x, Ws)
    return _sc_aggregate(y, src_all, dst_all, bbar)
